# baseline (speedup 1.0000x reference)
import functools

import jax
import jax.numpy as jnp
from jax import lax
from jax.experimental import pallas as pl
from jax.experimental.pallas import tpu as pltpu
from jax.experimental.pallas import tpu_sc as plsc

NC, NS = 2, 16
NW = NC * NS
BATCH, SEQ = 4096, 50
B = BATCH * SEQ
D = 128
ROWS_PER_W = B // NW
CHUNK = 128
G = ROWS_PER_W // CHUNK
NBUF = 5
LEAD = 3

_mesh = plsc.VectorSubcoreMesh(core_axis_name="c", subcore_axis_name="s")


@functools.partial(
    pl.kernel,
    out_type=jax.ShapeDtypeStruct((B, D), jnp.float32),
    mesh=_mesh,
    scratch_types=(
        [pltpu.VMEM((G, CHUNK), jnp.int32)]
        + [pltpu.VMEM((CHUNK, D), jnp.float32) for _ in range(NBUF)]
        + [pltpu.SemaphoreType.DMA for _ in range(2 * NBUF)]
    ),
    compiler_params=pltpu.CompilerParams(use_tc_tiling_on_sc=True),
)
def _embedding_gather(table_hbm, idx_hbm, out_hbm, idx_v, *scratch):
    bufs = scratch[:NBUF]
    gsem = scratch[NBUF:2 * NBUF]
    ssem = scratch[2 * NBUF:]
    wid = lax.axis_index("s") * NC + lax.axis_index("c")
    obase = wid * ROWS_PER_W
    pltpu.sync_copy(idx_hbm.at[wid], idx_v)

    def start_gather(g, b):
        pltpu.make_async_copy(table_hbm.at[idx_v.at[g]], bufs[b], gsem[b]).start()

    def wait_gather(b):
        pltpu.make_async_copy(table_hbm.at[pl.ds(0, CHUNK)], bufs[b], gsem[b]).wait()

    def start_store(g, b):
        pltpu.make_async_copy(
            bufs[b], out_hbm.at[pl.ds(obase + g * CHUNK, CHUNK)], ssem[b]
        ).start()

    def wait_store(b):
        pltpu.make_async_copy(
            bufs[b], out_hbm.at[pl.ds(obase, CHUNK)], ssem[b]
        ).wait()

    for b in range(LEAD):
        start_gather(b, b)

    def slot(g, b):
        wait_gather(b)
        start_store(g, b)
        gp = g + LEAD
        bp = (b + LEAD) % NBUF

        @pl.when(gp < G)
        def _prefetch():
            @pl.when(gp >= NBUF)
            def _drain():
                wait_store(bp)

            start_gather(gp, bp)

    def body(i, carry):
        for b in range(NBUF):
            slot(i * NBUF + b, b)
        return carry

    lax.fori_loop(0, G // NBUF, body, 0)

    for b in range(NBUF):
        wait_store(b)


def kernel(input_ids, weight):
    idx = input_ids.reshape(NW, G, CHUNK).astype(jnp.int32)
    out = _embedding_gather(weight, idx)
    return out.reshape(BATCH, SEQ, D)


# speedup vs baseline: 1.7840x; 1.7840x over previous
import functools

import jax
import jax.numpy as jnp
from jax import lax
from jax.experimental import pallas as pl
from jax.experimental.pallas import tpu as pltpu
from jax.experimental.pallas import tpu_sc as plsc

NC, NS = 2, 16
NW = NC * NS
BATCH, SEQ = 4096, 50
D = 128
SEQP = 56
RPW = BATCH // NW
GRP = 8
NG = RPW // GRP
NBUF = 2

_mesh = plsc.VectorSubcoreMesh(core_axis_name="c", subcore_axis_name="s")


@functools.partial(
    pl.kernel,
    out_type=jax.ShapeDtypeStruct((BATCH, SEQ, D), jnp.float32),
    mesh=_mesh,
    scratch_types=(
        [pltpu.VMEM((RPW, SEQP), jnp.int32)]
        + [pltpu.VMEM((GRP, SEQ, D), jnp.float32) for _ in range(NBUF)]
        + [pltpu.SemaphoreType.DMA for _ in range(2 * NBUF)]
    ),
    compiler_params=pltpu.CompilerParams(use_tc_tiling_on_sc=True),
)
def _embedding_gather(table_hbm, idx_hbm, out_hbm, idx_v, *scratch):
    bufs = scratch[:NBUF]
    gsem = scratch[NBUF:2 * NBUF]
    ssem = scratch[2 * NBUF:]
    wid = lax.axis_index("s") * NC + lax.axis_index("c")
    rbase = wid * RPW
    pltpu.sync_copy(idx_hbm.at[wid], idx_v)

    def fire_group(k, b):
        for j in range(GRP):
            pltpu.make_async_copy(
                table_hbm.at[idx_v.at[k * GRP + j, pl.ds(0, SEQ)]],
                bufs[b].at[j],
                gsem[b],
            ).start()

    def drain_group(b):
        pltpu.make_async_copy(
            out_hbm.at[pl.ds(rbase, GRP)],
            bufs[b],
            gsem[b],
        ).wait()

    def start_store(k, b):
        pltpu.make_async_copy(
            bufs[b], out_hbm.at[pl.ds(rbase + k * GRP, GRP)], ssem[b]
        ).start()

    def wait_store(b):
        pltpu.make_async_copy(
            bufs[b], out_hbm.at[pl.ds(rbase, GRP)], ssem[b]
        ).wait()

    fire_group(0, 0)

    def slot(k, b):
        kn = k + 1

        @pl.when(kn < NG)
        def _prefetch():
            @pl.when(kn >= NBUF)
            def _drain():
                wait_store(1 - b)

            fire_group(kn, 1 - b)

        drain_group(b)
        start_store(k, b)

    def body(i, carry):
        for b in range(NBUF):
            slot(i * NBUF + b, b)
        return carry

    lax.fori_loop(0, NG // NBUF, body, 0)

    for b in range(NBUF):
        wait_store(b)


def kernel(input_ids, weight):
    idx = input_ids.reshape(NW, RPW, SEQ).astype(jnp.int32)
    idx = jnp.pad(idx, ((0, 0), (0, 0), (0, SEQP - SEQ)))
    return _embedding_gather(weight, idx)


# speedup vs baseline: 3.2127x; 1.8008x over previous
import functools

import jax
import jax.numpy as jnp
from jax import lax
from jax.experimental import pallas as pl
from jax.experimental.pallas import tpu as pltpu
from jax.experimental.pallas import tpu_sc as plsc

NC, NS = 2, 16
NW = NC * NS
BATCH, SEQ = 4096, 50
D = 128
CHUNK = BATCH // NW
G = SEQ
NBUF = 5
LEAD = 3

_mesh = plsc.VectorSubcoreMesh(core_axis_name="c", subcore_axis_name="s")


@functools.partial(
    pl.kernel,
    out_type=jax.ShapeDtypeStruct((SEQ, BATCH, D), jnp.float32),
    mesh=_mesh,
    scratch_types=(
        [pltpu.VMEM((G, CHUNK), jnp.int32)]
        + [pltpu.VMEM((CHUNK, D), jnp.float32) for _ in range(NBUF)]
        + [pltpu.SemaphoreType.DMA for _ in range(2 * NBUF)]
    ),
    compiler_params=pltpu.CompilerParams(use_tc_tiling_on_sc=True),
)
def _embedding_gather(table_hbm, idx_hbm, out_hbm, idx_v, *scratch):
    bufs = scratch[:NBUF]
    gsem = scratch[NBUF:2 * NBUF]
    ssem = scratch[2 * NBUF:]
    wid = lax.axis_index("s") * NC + lax.axis_index("c")
    bbase = wid * CHUNK
    pltpu.sync_copy(idx_hbm.at[:, pl.ds(bbase, CHUNK)], idx_v)

    def start_gather(g, b):
        pltpu.make_async_copy(table_hbm.at[idx_v.at[g]], bufs[b], gsem[b]).start()

    def wait_gather(b):
        pltpu.make_async_copy(table_hbm.at[pl.ds(0, CHUNK)], bufs[b], gsem[b]).wait()

    def start_store(g, b):
        pltpu.make_async_copy(
            bufs[b], out_hbm.at[g, pl.ds(bbase, CHUNK)], ssem[b]
        ).start()

    def wait_store(b):
        pltpu.make_async_copy(
            bufs[b], out_hbm.at[0, pl.ds(bbase, CHUNK)], ssem[b]
        ).wait()

    for b in range(LEAD):
        start_gather(b, b)

    def slot(g, b):
        wait_gather(b)
        start_store(g, b)
        gp = g + LEAD
        bp = (b + LEAD) % NBUF

        @pl.when(gp < G)
        def _prefetch():
            @pl.when(gp >= NBUF)
            def _drain():
                wait_store(bp)

            start_gather(gp, bp)

    def body(i, carry):
        for b in range(NBUF):
            slot(i * NBUF + b, b)
        return carry

    lax.fori_loop(0, G // NBUF, body, 0)

    for b in range(NBUF):
        wait_store(b)


def kernel(input_ids, weight):
    idx_t = input_ids.astype(jnp.int32).T
    out = _embedding_gather(weight, idx_t)
    return out.transpose(1, 0, 2)
